# baseline (speedup 1.0000x reference)
import jax
import jax.numpy as jnp
import numpy as np
from jax.experimental import pallas as pl
from jax.experimental.pallas import tpu as pltpu

_BS = 256
_TLEN = 16
_NN = 16
_D2 = 16
_HID = 8
_EPS = 1e-5
_DECAY = 0.7
_NEG = 0.01


def _leaky(x):
    return jnp.where(x >= 0, x, _NEG * x)


def _eye(n, dtype=jnp.float32):
    r = jax.lax.broadcasted_iota(jnp.int32, (n, n), 0)
    c = jax.lax.broadcasted_iota(jnp.int32, (n, n), 1)
    return (r == c).astype(dtype)


def _window(gram, base, n, w, mask, xb, tT, tb, ms, mb):
    s = jax.lax.slice(gram, (base, base), (base + n, base + n))
    eye = _eye(n)
    s = _leaky(s - eye * 1e8)
    m = jnp.max(s, axis=-1, keepdims=True)
    e = jnp.exp(s - m)
    adj = e / jnp.sum(e, axis=-1, keepdims=True)
    adj = (adj + eye) * mask
    xw = jax.lax.slice(xb, (base, 0), (base + n, _D2))
    h = jnp.dot(adj, xw, preferred_element_type=jnp.float32)
    h = jnp.dot(h, tT, preferred_element_type=jnp.float32) + tb
    h = _leaky(h * ms + mb)
    acc = h[0:_NN]
    for k in range(1, w):
        acc = acc + h[k * _NN:(k + 1) * _NN]
    return acc * (1.0 / w)


def _body(xu_ref, pe_ref, m1_ref, b1_ref, m2_ref, b2_ref, l2_ref, l2b_ref,
          g1t_ref, g1b_ref, g2t_ref, g2b_ref,
          s1_ref, sb1_ref, s2_ref, sb2_ref,
          t1t_ref, t1b_ref, m1s_ref, m1b_ref,
          t2t_ref, t2b_ref, m2s_ref, m2b_ref,
          mask1_ref, mask2_ref, w1_ref, fb1_ref,
          f2t_ref, fb2_ref, f3t_ref, fb3_ref, f4_ref, fb4_ref,
          out_ref):
    x = xu_ref[...]
    h = jnp.maximum(jnp.dot(x, m1_ref[...], preferred_element_type=jnp.float32)
                    + b1_ref[...], 0.0)
    h = jnp.maximum(jnp.dot(h, m2_ref[...], preferred_element_type=jnp.float32)
                    + b2_ref[...], 0.0)
    a4 = (jnp.dot(h, l2_ref[...], preferred_element_type=jnp.float32)
          + l2b_ref[...] + pe_ref[...])

    nf1 = jnp.dot(a4, g1t_ref[...], preferred_element_type=jnp.float32) + g1b_ref[...]
    nf2 = jnp.dot(a4, g2t_ref[...], preferred_element_type=jnp.float32) + g2b_ref[...]
    dn = (((1,), (1,)), ((), ()))
    gram1 = jax.lax.dot_general(nf1, nf1, dn, preferred_element_type=jnp.float32)
    gram2 = jax.lax.dot_general(nf2, nf2, dn, preferred_element_type=jnp.float32)

    xb1 = a4 * s1_ref[...] + sb1_ref[...]
    xb2 = a4 * s2_ref[...] + sb2_ref[...]

    pieces = []
    for j in range(4):
        pieces.append(_window(gram1, 64 * j, 64, 4, mask1_ref[...], xb1,
                              t1t_ref[...], t1b_ref[...], m1s_ref[...], m1b_ref[...]))
    for j in range(3):
        pieces.append(_window(gram2, 64 * j, 128, 8, mask2_ref[...], xb2,
                              t2t_ref[...], t2b_ref[...], m2s_ref[...], m2b_ref[...]))
    hcat = jnp.concatenate(pieces, axis=0)

    f = jnp.sum(hcat[:, :, None] * w1_ref[...], axis=(0, 1))[None, :]
    f = jnp.maximum(f + fb1_ref[...], 0.0)
    f = jnp.maximum(jnp.dot(f, f2t_ref[...], preferred_element_type=jnp.float32)
                    + fb2_ref[...], 0.0)
    f = jnp.maximum(jnp.dot(f, f3t_ref[...], preferred_element_type=jnp.float32)
                    + fb3_ref[...], 0.0)
    y = jnp.sum(f * f4_ref[...]) + fb4_ref[0, 0]
    out_ref[...] = jnp.full((1, 8, 128), y, dtype=jnp.float32)


@jax.jit
def kernel(X, params):
    p = params
    f32 = jnp.float32
    bs = X.shape[0]

    xu = jnp.transpose(X.reshape(bs, _TLEN, 16, _NN), (0, 1, 3, 2))
    xu = xu.reshape(bs * _TLEN * _NN, 16)

    ti = jnp.arange(16)[:, None] - jnp.arange(16)[None, :]
    bands = jnp.stack([(ti == k - 1).astype(f32) for k in range(3)])
    s_c1 = p['bn_c1_g'] / jnp.sqrt(1.0 + _EPS)
    w1c = p['conv1_w'][:, 0, :] * s_c1[:, None]
    m1 = jnp.einsum('ck,ktu->ctu', w1c, bands)
    m1 = jnp.transpose(m1, (1, 0, 2)).reshape(16, 128)
    b1 = jnp.repeat(p['bn_c1_b'], 16)[None, :]

    s_c2 = p['bn_c2_g'] / jnp.sqrt(1.0 + _EPS)
    w2c = p['conv2_w'] * s_c2[:, None, None]
    m2 = jnp.einsum('oik,ktu->itou', w2c, bands)
    m2 = m2.reshape(128, 128)
    b2 = jnp.repeat(p['bn_c2_b'], 16)[None, :]

    s_l = p['bn2_g'] / jnp.sqrt(1.0 + _EPS)
    l2 = p['lin2_w'].T * s_l[None, :]
    l2b = (p['lin2_b'] * s_l + p['bn2_b'])[None, :]

    pos = jnp.arange(_TLEN, dtype=f32)[:, None]
    div = jnp.exp(jnp.arange(0, _D2, 2, dtype=f32) * (-np.log(10000.0) / _D2))
    pe = jnp.zeros((_TLEN, _D2), f32)
    pe = pe.at[:, 0::2].set(jnp.sin(pos * div))
    pe = pe.at[:, 1::2].set(jnp.cos(pos * div))
    pe_exp = jnp.repeat(pe, _NN, axis=0)

    def bn_pair(g, b):
        s = g / jnp.sqrt(1.0 + _EPS)
        return s[None, :], b[None, :]

    s1, sb1 = bn_pair(p['bnb1_g'], p['bnb1_b'])
    s2, sb2 = bn_pair(p['bnb2_g'], p['bnb2_b'])
    m1s, m1b = bn_pair(p['bnm1_g'], p['bnm1_b'])
    m2s, m2b = bn_pair(p['bnm2_g'], p['bnm2_b'])

    def decay_mask(w):
        t = jnp.arange(w * _NN) // _NN
        return (_DECAY ** jnp.abs(t[:, None] - t[None, :]).astype(f32)).astype(f32)

    mask1 = decay_mask(4)
    mask2 = decay_mask(8)

    w1 = jnp.transpose(p['fc1_w'].reshape(_D2, 7 * _NN, _HID), (1, 2, 0))
    fb1 = p['fc1_b'][None, :]
    f2t = p['fc2_w'].T
    fb2 = p['fc2_b'][None, :]
    f3t = p['fc3_w'].T
    fb3 = p['fc3_b'][None, :]
    f4 = p['fc4_w'][0][None, :]
    fb4 = p['fc4_b'][None, :]

    full = lambda shp: pl.BlockSpec(shp, lambda b: tuple(0 for _ in shp))
    in_specs = [
        pl.BlockSpec((_TLEN * _NN, 16), lambda b: (b, 0)),
        full((256, 16)),
        full((16, 128)), full((1, 128)),
        full((128, 128)), full((1, 128)),
        full((128, 16)), full((1, 16)),
        full((16, 16)), full((1, 16)),
        full((16, 16)), full((1, 16)),
        full((1, 16)), full((1, 16)),
        full((1, 16)), full((1, 16)),
        full((16, 8)), full((1, 8)),
        full((1, 8)), full((1, 8)),
        full((16, 8)), full((1, 8)),
        full((1, 8)), full((1, 8)),
        full((64, 64)), full((128, 128)),
        full((112, 8, 16)), full((1, 16)),
        full((16, 16)), full((1, 16)),
        full((16, 8)), full((1, 8)),
        full((1, 8)), full((1, 1)),
    ]
    out = pl.pallas_call(
        _body,
        grid=(bs,),
        in_specs=in_specs,
        out_specs=pl.BlockSpec((1, 8, 128), lambda b: (b, 0, 0)),
        out_shape=jax.ShapeDtypeStruct((bs, 8, 128), f32),
        compiler_params=pltpu.CompilerParams(
            dimension_semantics=("arbitrary",),
        ),
    )(xu, pe_exp, m1, b1, m2, b2, l2, l2b,
      p['g1_w'].T, p['g1_b'][None, :],
      p['g2_w'].T, p['g2_b'][None, :],
      s1, sb1, s2, sb2,
      p['t1_w'].T, p['t1_b'][None, :], m1s, m1b,
      p['t2_w'].T, p['t2_b'][None, :], m2s, m2b,
      mask1, mask2, w1, fb1, f2t, fb2, f3t, fb3, f4, fb4)
    return out[:, 0, :1]
